# baseline (speedup 1.0000x reference)
import functools

import jax
import jax.numpy as jnp
from jax.experimental import pallas as pl
from jax.experimental.pallas import tpu as pltpu

B = 4096
D = 1024
K = 64
N_SAMPLES = 1000
GRID = 8
BLOCK = B // GRID


def _mix_kernel(u_hbm, cw_hbm, mus_hbm, lv_hbm, x_ref, elbo_ref, loss_ref,
                u_v, cw_v, mus_v, lv_v, wiv_s, wmiv_s, c_s, acc_s, sem):
    i = pl.program_id(0)

    @pl.when(i == 0)
    def _prologue():
        for src, dst in ((u_hbm, u_v), (cw_hbm, cw_v),
                         (mus_hbm, mus_v), (lv_hbm, lv_v)):
            pltpu.make_async_copy(src, dst, sem).start()
        for src, dst in ((u_hbm, u_v), (cw_hbm, cw_v),
                         (mus_hbm, mus_v), (lv_hbm, lv_v)):
            pltpu.make_async_copy(src, dst, sem).wait()

        u = u_v[:]
        g = -jnp.log(-jnp.log(u)) + cw_v[:]
        rowmax = jnp.max(g, axis=1, keepdims=True)
        col = jax.lax.broadcasted_iota(jnp.int32, g.shape, 1)
        idx = jnp.where(g == rowmax, col, K)
        amin = jnp.min(idx, axis=1, keepdims=True)
        firsthot = (col == amin).astype(jnp.float32)
        w = jnp.sum(firsthot, axis=0, keepdims=True) / N_SAMPLES

        lv = lv_v[:]
        iv = jnp.exp(-lv)
        mus = mus_v[:]
        dot = functools.partial(jax.lax.dot_general,
                                dimension_numbers=(((1,), (0,)), ((), ())),
                                precision=jax.lax.Precision.HIGHEST,
                                preferred_element_type=jnp.float32)
        wiv_s[:] = dot(w, iv)
        wmiv_s[:] = 2.0 * dot(w, mus * iv)
        t = jnp.sum(mus * mus * iv + lv, axis=1, keepdims=True)
        c_s[0, 0] = dot(w, t)[0, 0] + D * jnp.log(2.0 * jnp.pi)
        acc_s[0, 0] = 0.0

    xb = x_ref[:]
    row = jnp.sum(xb * (xb * wiv_s[:] - wmiv_s[:]), axis=1)
    elbo_ref[:] = (-0.5 * (row + c_s[0, 0])).reshape(BLOCK, 1)
    acc_s[0, 0] += jnp.sum(row)

    @pl.when(i == GRID - 1)
    def _epilogue():
        loss_ref[:] = jnp.full((1, 1), 0.5 * (acc_s[0, 0] / B + c_s[0, 0]),
                               dtype=jnp.float32)


def kernel(x, categorical_weights, mus, log_var):
    key = jax.random.key(42)
    u = jax.random.uniform(key, (N_SAMPLES, K), jnp.float32,
                           minval=jnp.finfo(jnp.float32).tiny, maxval=1.0)
    cw = categorical_weights.reshape(1, K)

    elbo2d, loss2d = pl.pallas_call(
        _mix_kernel,
        grid=(GRID,),
        in_specs=[
            pl.BlockSpec(memory_space=pl.ANY),
            pl.BlockSpec(memory_space=pl.ANY),
            pl.BlockSpec(memory_space=pl.ANY),
            pl.BlockSpec(memory_space=pl.ANY),
            pl.BlockSpec((BLOCK, D), lambda i: (i, 0)),
        ],
        out_specs=[
            pl.BlockSpec((BLOCK, 1), lambda i: (i, 0)),
            pl.BlockSpec((1, 1), lambda i: (0, 0)),
        ],
        out_shape=[
            jax.ShapeDtypeStruct((B, 1), jnp.float32),
            jax.ShapeDtypeStruct((1, 1), jnp.float32),
        ],
        scratch_shapes=[
            pltpu.VMEM((N_SAMPLES, K), jnp.float32),
            pltpu.VMEM((1, K), jnp.float32),
            pltpu.VMEM((K, D), jnp.float32),
            pltpu.VMEM((K, D), jnp.float32),
            pltpu.VMEM((1, D), jnp.float32),
            pltpu.VMEM((1, D), jnp.float32),
            pltpu.SMEM((1, 1), jnp.float32),
            pltpu.SMEM((1, 1), jnp.float32),
            pltpu.SemaphoreType.DMA,
        ],
    )(u, cw, mus, log_var, x)

    return loss2d[0, 0], elbo2d[:, 0]


# speedup vs baseline: 1.1164x; 1.1164x over previous
import functools

import jax
import jax.numpy as jnp
from jax.experimental import pallas as pl
from jax.experimental.pallas import tpu as pltpu

B = 4096
D = 1024
K = 64
N_SAMPLES = 1000
GRID = 2
BLOCK = B // GRID


def _mix_kernel(u_hbm, cw_hbm, mus_hbm, lv_hbm, x_ref, elbo_ref, loss_ref,
                u_v, cw_v, mus_v, lv_v, wiv_s, wmiv_s, c_s, acc_s, sem):
    i = pl.program_id(0)

    @pl.when(i == 0)
    def _prologue():
        for src, dst in ((u_hbm, u_v), (cw_hbm, cw_v),
                         (mus_hbm, mus_v), (lv_hbm, lv_v)):
            pltpu.make_async_copy(src, dst, sem).start()
        for src, dst in ((u_hbm, u_v), (cw_hbm, cw_v),
                         (mus_hbm, mus_v), (lv_hbm, lv_v)):
            pltpu.make_async_copy(src, dst, sem).wait()

        u = u_v[:]
        g = -jnp.log(-jnp.log(u)) + cw_v[:]
        rowmax = jnp.max(g, axis=1, keepdims=True)
        col = jax.lax.broadcasted_iota(jnp.int32, g.shape, 1)
        idx = jnp.where(g == rowmax, col, K)
        amin = jnp.min(idx, axis=1, keepdims=True)
        firsthot = (col == amin).astype(jnp.float32)
        w = jnp.sum(firsthot, axis=0, keepdims=True) / N_SAMPLES

        lv = lv_v[:]
        iv = jnp.exp(-lv)
        mus = mus_v[:]
        dot = functools.partial(jax.lax.dot_general,
                                dimension_numbers=(((1,), (0,)), ((), ())),
                                precision=jax.lax.Precision.HIGHEST,
                                preferred_element_type=jnp.float32)
        wiv_s[:] = dot(w, iv)
        wmiv_s[:] = 2.0 * dot(w, mus * iv)
        t = jnp.sum(mus * mus * iv + lv, axis=1, keepdims=True)
        c_s[0, 0] = dot(w, t)[0, 0] + D * jnp.log(2.0 * jnp.pi)
        acc_s[0, 0] = 0.0

    xb = x_ref[:]
    row = jnp.sum(xb * (xb * wiv_s[:] - wmiv_s[:]), axis=1)
    elbo_ref[:] = (-0.5 * (row + c_s[0, 0])).reshape(BLOCK, 1)
    acc_s[0, 0] += jnp.sum(row)

    @pl.when(i == GRID - 1)
    def _epilogue():
        loss_ref[:] = jnp.full((1, 1), 0.5 * (acc_s[0, 0] / B + c_s[0, 0]),
                               dtype=jnp.float32)


def kernel(x, categorical_weights, mus, log_var):
    key = jax.random.key(42)
    u = jax.random.uniform(key, (N_SAMPLES, K), jnp.float32,
                           minval=jnp.finfo(jnp.float32).tiny, maxval=1.0)
    cw = categorical_weights.reshape(1, K)

    elbo2d, loss2d = pl.pallas_call(
        _mix_kernel,
        grid=(GRID,),
        in_specs=[
            pl.BlockSpec(memory_space=pl.ANY),
            pl.BlockSpec(memory_space=pl.ANY),
            pl.BlockSpec(memory_space=pl.ANY),
            pl.BlockSpec(memory_space=pl.ANY),
            pl.BlockSpec((BLOCK, D), lambda i: (i, 0)),
        ],
        out_specs=[
            pl.BlockSpec((BLOCK, 1), lambda i: (i, 0)),
            pl.BlockSpec((1, 1), lambda i: (0, 0)),
        ],
        out_shape=[
            jax.ShapeDtypeStruct((B, 1), jnp.float32),
            jax.ShapeDtypeStruct((1, 1), jnp.float32),
        ],
        scratch_shapes=[
            pltpu.VMEM((N_SAMPLES, K), jnp.float32),
            pltpu.VMEM((1, K), jnp.float32),
            pltpu.VMEM((K, D), jnp.float32),
            pltpu.VMEM((K, D), jnp.float32),
            pltpu.VMEM((1, D), jnp.float32),
            pltpu.VMEM((1, D), jnp.float32),
            pltpu.SMEM((1, 1), jnp.float32),
            pltpu.SMEM((1, 1), jnp.float32),
            pltpu.SemaphoreType.DMA,
        ],
    )(u, cw, mus, log_var, x)

    return loss2d[0, 0], elbo2d[:, 0]


# speedup vs baseline: 1.2695x; 1.1371x over previous
import functools

import jax
import jax.numpy as jnp
from jax.experimental import pallas as pl
from jax.experimental.pallas import tpu as pltpu

B = 4096
D = 1024
K = 64
N_SAMPLES = 1000
GRID = 2
BLOCK = B // GRID


def _mix_kernel(u_ref, cw_ref, mus_ref, lv_ref, x_ref, elbo_ref, loss_ref,
                wiv_s, wmiv_s, c_s, acc_s):
    i = pl.program_id(0)

    @pl.when(i == 0)
    def _prologue():
        u = u_ref[:]
        g = -jnp.log(-jnp.log(u)) + cw_ref[:]
        rowmax = jnp.max(g, axis=1, keepdims=True)
        col = jax.lax.broadcasted_iota(jnp.int32, g.shape, 1)
        idx = jnp.where(g == rowmax, col, K)
        amin = jnp.min(idx, axis=1, keepdims=True)
        firsthot = (col == amin).astype(jnp.float32)
        w = jnp.sum(firsthot, axis=0, keepdims=True) / N_SAMPLES

        lv = lv_ref[:]
        iv = jnp.exp(-lv)
        mus = mus_ref[:]
        dot = functools.partial(jax.lax.dot_general,
                                dimension_numbers=(((1,), (0,)), ((), ())),
                                precision=jax.lax.Precision.HIGHEST,
                                preferred_element_type=jnp.float32)
        wiv_s[:] = dot(w, iv)
        wmiv_s[:] = 2.0 * dot(w, mus * iv)
        t = jnp.sum(mus * mus * iv + lv, axis=1, keepdims=True)
        c_s[0, 0] = dot(w, t)[0, 0] + D * jnp.log(2.0 * jnp.pi)
        acc_s[0, 0] = 0.0

    xb = x_ref[:]
    row = jnp.sum(xb * (xb * wiv_s[:] - wmiv_s[:]), axis=1)
    elbo_ref[:] = (-0.5 * (row + c_s[0, 0])).reshape(BLOCK, 1)
    acc_s[0, 0] += jnp.sum(row)

    @pl.when(i == GRID - 1)
    def _epilogue():
        loss_ref[:] = jnp.full((1, 1), 0.5 * (acc_s[0, 0] / B + c_s[0, 0]),
                               dtype=jnp.float32)


def kernel(x, categorical_weights, mus, log_var):
    key = jax.random.key(42)
    u = jax.random.uniform(key, (N_SAMPLES, K), jnp.float32,
                           minval=jnp.finfo(jnp.float32).tiny, maxval=1.0)
    cw = categorical_weights.reshape(1, K)

    elbo2d, loss2d = pl.pallas_call(
        _mix_kernel,
        grid=(GRID,),
        in_specs=[
            pl.BlockSpec((N_SAMPLES, K), lambda i: (0, 0)),
            pl.BlockSpec((1, K), lambda i: (0, 0)),
            pl.BlockSpec((K, D), lambda i: (0, 0)),
            pl.BlockSpec((K, D), lambda i: (0, 0)),
            pl.BlockSpec((BLOCK, D), lambda i: (i, 0)),
        ],
        out_specs=[
            pl.BlockSpec((BLOCK, 1), lambda i: (i, 0)),
            pl.BlockSpec((1, 1), lambda i: (0, 0)),
        ],
        out_shape=[
            jax.ShapeDtypeStruct((B, 1), jnp.float32),
            jax.ShapeDtypeStruct((1, 1), jnp.float32),
        ],
        scratch_shapes=[
            pltpu.VMEM((1, D), jnp.float32),
            pltpu.VMEM((1, D), jnp.float32),
            pltpu.SMEM((1, 1), jnp.float32),
            pltpu.SMEM((1, 1), jnp.float32),
        ],
    )(u, cw, mus, log_var, x)

    return loss2d[0, 0], elbo2d[:, 0]
